# Initial kernel scaffold; baseline (speedup 1.0000x reference)
import jax
import jax.numpy as jnp
from jax.experimental import pallas as pl


def kernel(in_feat, edge_index, W_self1, W_neigh1, b1, W_self2, W_neigh2, b2):
    raise NotImplementedError("write your pallas kernel here")


# speedup vs baseline: 3.0914x; 3.0914x over previous
import functools

import jax
import jax.numpy as jnp
from jax import lax
from jax.experimental import pallas as pl
from jax.experimental.pallas import tpu as pltpu
from jax.experimental.pallas import tpu_sc as plsc

NC = 2
NS = 16
NW = NC * NS
LANES = 16
CHUNK = 128
BLKC = 16


@functools.lru_cache(maxsize=None)
def _edge_agg(n_nodes, d, n_chunks, want_deg):
    npad = ((n_nodes + 1 + 127) // 128) * 128
    zrows = npad // NS
    assert n_chunks % BLKC == 0
    zsegs = []
    r0 = 0
    while r0 < zrows:
        zsegs.append((r0, min(CHUNK, zrows - r0)))
        r0 += CHUNK

    out_type = [jax.ShapeDtypeStruct((NC, npad, d), jnp.float32)]
    if want_deg:
        out_type.append(jax.ShapeDtypeStruct((NC, npad, d), jnp.float32))
    scratch = [
        pltpu.VMEM((BLKC, CHUNK), jnp.int32),
        pltpu.VMEM((BLKC, CHUNK), jnp.int32),
        pltpu.VMEM((CHUNK, d), jnp.float32),
        pltpu.VMEM_SHARED((npad, d), jnp.float32),
        pltpu.SemaphoreType.DMA,
    ]

    mesh = plsc.VectorSubcoreMesh(
        core_axis_name="c", subcore_axis_name="s", num_cores=NC,
        num_subcores=NS)

    def body(y_hbm, srcr_hbm, dstr_hbm, out_hbm, *rest):
        if want_deg:
            deg_hbm, src_v, dst_v, rows_v, acc, sem = rest
        else:
            src_v, dst_v, rows_v, acc, sem = rest
        c = lax.axis_index("c")
        s = lax.axis_index("s")
        w = c * NS + s

        def fill_rows(val):
            vv = jnp.full((LANES,), val, jnp.float32)

            def _row(i, _):
                for j in range(d // LANES):
                    rows_v[i, pl.ds(j * LANES, LANES)] = vv
                return 0

            lax.fori_loop(0, CHUNK, _row, 0)

        def zero_acc():
            for (off, nr) in zsegs:
                pltpu.sync_copy(rows_v.at[pl.ds(0, nr)],
                                acc.at[pl.ds(s * zrows + off, nr)])

        def copy_out(dst_hbm):
            pltpu.sync_copy(acc.at[pl.ds(s * zrows, zrows)],
                            dst_hbm.at[c, pl.ds(s * zrows, zrows)])

        fill_rows(0.0)
        zero_acc()

        if want_deg:
            plsc.subcore_barrier()
            fill_rows(1.0)

            def dstage(b, _):
                pltpu.sync_copy(dstr_hbm.at[w, pl.ds(b * BLKC, BLKC)], dst_v)

                def dstep(j, _):
                    pltpu.sync_copy(rows_v, acc.at[dst_v.at[j]], add=True)
                    return 0

                lax.fori_loop(0, BLKC, dstep, 0)
                return 0

            lax.fori_loop(0, n_chunks // BLKC, dstage, 0)
            plsc.subcore_barrier()
            copy_out(deg_hbm)
            fill_rows(0.0)
            zero_acc()

        plsc.subcore_barrier()

        def stage(b, _):
            pltpu.sync_copy(srcr_hbm.at[w, pl.ds(b * BLKC, BLKC)], src_v)
            pltpu.sync_copy(dstr_hbm.at[w, pl.ds(b * BLKC, BLKC)], dst_v)

            def step(j, _):
                pltpu.async_copy(y_hbm.at[src_v.at[j]], rows_v, sem).wait()
                pltpu.sync_copy(rows_v, acc.at[dst_v.at[j]], add=True)
                return 0

            lax.fori_loop(0, BLKC, step, 0)
            return 0

        lax.fori_loop(0, n_chunks // BLKC, stage, 0)
        plsc.subcore_barrier()
        copy_out(out_hbm)

    return pl.kernel(body, out_type=out_type, mesh=mesh, scratch_types=scratch)


_BLK = 1000


def _proj2_body(x_ref, wa_ref, wb_ref, oa_ref, ob_ref):
    x = x_ref[...]
    oa_ref[...] = jnp.dot(x, wa_ref[...], preferred_element_type=jnp.float32)
    ob_ref[...] = jnp.dot(x, wb_ref[...], preferred_element_type=jnp.float32)


def _proj2(x, wa, wb):
    n, d = x.shape
    h = wa.shape[1]
    return pl.pallas_call(
        _proj2_body,
        grid=(n // _BLK,),
        in_specs=[
            pl.BlockSpec((_BLK, d), lambda i: (i, 0)),
            pl.BlockSpec((d, h), lambda i: (0, 0)),
            pl.BlockSpec((d, h), lambda i: (0, 0)),
        ],
        out_specs=[
            pl.BlockSpec((_BLK, h), lambda i: (i, 0)),
            pl.BlockSpec((_BLK, h), lambda i: (i, 0)),
        ],
        out_shape=[jax.ShapeDtypeStruct((n, h), jnp.float32)] * 2,
    )(x, wa, wb)


def _mid_body(s1_ref, p_ref, degp_ref, b1_ref, wa_ref, wb_ref, s2_ref, y2_ref):
    deg = degp_ref[0, :, 0] + degp_ref[1, :, 0]
    inv = 1.0 / jnp.maximum(deg, 1.0)
    agg = (p_ref[0] + p_ref[1]) * inv[:, None]
    h1 = jnp.maximum(s1_ref[...] + agg + b1_ref[...], 0.0)
    s2_ref[...] = jnp.dot(h1, wa_ref[...], preferred_element_type=jnp.float32)
    y2_ref[...] = jnp.dot(h1, wb_ref[...], preferred_element_type=jnp.float32)


def _mid(s1, p, degp, b1, wa, wb):
    n, h = s1.shape
    return pl.pallas_call(
        _mid_body,
        grid=(n // _BLK,),
        in_specs=[
            pl.BlockSpec((_BLK, h), lambda i: (i, 0)),
            pl.BlockSpec((NC, _BLK, h), lambda i: (0, i, 0)),
            pl.BlockSpec((NC, _BLK, h), lambda i: (0, i, 0)),
            pl.BlockSpec((1, h), lambda i: (0, 0)),
            pl.BlockSpec((h, h), lambda i: (0, 0)),
            pl.BlockSpec((h, h), lambda i: (0, 0)),
        ],
        out_specs=[
            pl.BlockSpec((_BLK, h), lambda i: (i, 0)),
            pl.BlockSpec((_BLK, h), lambda i: (i, 0)),
        ],
        out_shape=[jax.ShapeDtypeStruct((n, h), jnp.float32)] * 2,
    )(s1, p, degp, b1[None, :], wa, wb)


def _fin_body(s2_ref, q_ref, degp_ref, b2_ref, out_ref):
    deg = degp_ref[0, :, 0] + degp_ref[1, :, 0]
    inv = 1.0 / jnp.maximum(deg, 1.0)
    out_ref[...] = s2_ref[...] + (q_ref[0] + q_ref[1]) * inv[:, None] + b2_ref[...]


def _fin(s2, q, degp, b2):
    n, h = s2.shape
    return pl.pallas_call(
        _fin_body,
        grid=(n // _BLK,),
        in_specs=[
            pl.BlockSpec((_BLK, h), lambda i: (i, 0)),
            pl.BlockSpec((NC, _BLK, h), lambda i: (0, i, 0)),
            pl.BlockSpec((NC, _BLK, h), lambda i: (0, i, 0)),
            pl.BlockSpec((1, h), lambda i: (0, 0)),
        ],
        out_specs=pl.BlockSpec((_BLK, h), lambda i: (i, 0)),
        out_shape=jax.ShapeDtypeStruct((n, h), jnp.float32),
    )(s2, q, degp, b2[None, :])


def kernel(in_feat, edge_index, W_self1, W_neigh1, b1, W_self2, W_neigh2, b2):
    n, d = in_feat.shape
    e = edge_index.shape[1]
    ept = NW * CHUNK
    n_chunks = -(-e // ept)
    n_chunks = -(-n_chunks // BLKC) * BLKC
    e_pad = n_chunks * ept

    src = jnp.pad(edge_index[0], (0, e_pad - e))
    dst = jnp.pad(edge_index[1], (0, e_pad - e), constant_values=n)
    srcr = src.reshape(NW, n_chunks, CHUNK)
    dstr = dst.reshape(NW, n_chunks, CHUNK)

    s1, y1 = _proj2(in_feat, W_self1, W_neigh1)
    p, degp = _edge_agg(n, d, n_chunks, True)(y1, srcr, dstr)
    s2, y2 = _mid(s1, p, degp, b1, W_self2, W_neigh2)
    (q,) = _edge_agg(n, d, n_chunks, False)(y2, srcr, dstr)
    return _fin(s2, q, degp, b2)


# speedup vs baseline: 3.2765x; 1.0599x over previous
import functools

import jax
import jax.numpy as jnp
from jax import lax
from jax.experimental import pallas as pl
from jax.experimental.pallas import tpu as pltpu
from jax.experimental.pallas import tpu_sc as plsc

NC = 2
NS = 16
NW = NC * NS
LANES = 16
CHUNK = 128
BLKC = 16


@functools.lru_cache(maxsize=None)
def _edge_agg(n_nodes, d, n_chunks, want_deg):
    npad = ((n_nodes + 1 + 127) // 128) * 128
    zrows = npad // NS
    assert n_chunks % BLKC == 0
    zsegs = []
    r0 = 0
    while r0 < zrows:
        zsegs.append((r0, min(CHUNK, zrows - r0)))
        r0 += CHUNK

    out_type = [jax.ShapeDtypeStruct((NC, npad, d), jnp.float32)]
    if want_deg:
        out_type.append(jax.ShapeDtypeStruct((NC, npad, d), jnp.float32))
    scratch = [
        pltpu.VMEM((BLKC, CHUNK), jnp.int32),
        pltpu.VMEM((BLKC, CHUNK), jnp.int32),
        pltpu.VMEM((CHUNK, d), jnp.float32),
        pltpu.VMEM((CHUNK, d), jnp.float32),
        pltpu.VMEM_SHARED((npad, d), jnp.float32),
        pltpu.SemaphoreType.DMA,
        pltpu.SemaphoreType.DMA,
        pltpu.SemaphoreType.DMA,
        pltpu.SemaphoreType.DMA,
    ]

    mesh = plsc.VectorSubcoreMesh(
        core_axis_name="c", subcore_axis_name="s", num_cores=NC,
        num_subcores=NS)

    def body(y_hbm, srcr_hbm, dstr_hbm, out_hbm, *rest):
        if want_deg:
            deg_hbm, src_v, dst_v, rows_a, rows_b, acc, sga, sgb, ssa, ssb = rest
        else:
            src_v, dst_v, rows_a, rows_b, acc, sga, sgb, ssa, ssb = rest
        c = lax.axis_index("c")
        s = lax.axis_index("s")
        w = c * NS + s

        def fill_a(val):
            vv = jnp.full((LANES,), val, jnp.float32)

            def _row(i, _):
                for j in range(d // LANES):
                    rows_a[i, pl.ds(j * LANES, LANES)] = vv
                return 0

            lax.fori_loop(0, CHUNK, _row, 0)

        def zero_acc():
            for (off, nr) in zsegs:
                pltpu.sync_copy(rows_a.at[pl.ds(0, nr)],
                                acc.at[pl.ds(s * zrows + off, nr)])

        def copy_out(dst_hbm):
            pltpu.sync_copy(acc.at[pl.ds(s * zrows, zrows)],
                            dst_hbm.at[c, pl.ds(s * zrows, zrows)])

        def wait_gather(buf, sem):
            pltpu.make_async_copy(y_hbm.at[src_v.at[0]], buf, sem).wait()

        def wait_scatter(buf, sem):
            pltpu.make_async_copy(buf, acc.at[dst_v.at[0]], sem).wait()

        fill_a(0.0)
        zero_acc()

        if want_deg:
            plsc.subcore_barrier()
            fill_a(1.0)

            def dstage(b, _):
                pltpu.sync_copy(dstr_hbm.at[w, pl.ds(b * BLKC, BLKC)], dst_v)

                def dgroup(g, _):
                    for k in range(8):
                        pltpu.async_copy(
                            rows_a, acc.at[dst_v.at[g * 8 + k]], ssa, add=True)
                    for k in range(8):
                        wait_scatter(rows_a, ssa)
                    return 0

                lax.fori_loop(0, BLKC // 8, dgroup, 0)
                return 0

            lax.fori_loop(0, n_chunks // BLKC, dstage, 0)
            plsc.subcore_barrier()
            copy_out(deg_hbm)
            fill_a(0.0)
            zero_acc()

        plsc.subcore_barrier()

        def stage(b, _):
            pltpu.sync_copy(srcr_hbm.at[w, pl.ds(b * BLKC, BLKC)], src_v)
            pltpu.sync_copy(dstr_hbm.at[w, pl.ds(b * BLKC, BLKC)], dst_v)
            pltpu.async_copy(y_hbm.at[src_v.at[0]], rows_a, sga)
            pltpu.async_copy(y_hbm.at[src_v.at[1]], rows_b, sgb)

            def pair(i, _):
                j = 2 * i
                wait_gather(rows_a, sga)
                pltpu.async_copy(rows_a, acc.at[dst_v.at[j]], ssa, add=True)
                wait_gather(rows_b, sgb)
                pltpu.async_copy(rows_b, acc.at[dst_v.at[j + 1]], ssb, add=True)
                wait_scatter(rows_a, ssa)
                pltpu.async_copy(y_hbm.at[src_v.at[j + 2]], rows_a, sga)
                wait_scatter(rows_b, ssb)
                pltpu.async_copy(y_hbm.at[src_v.at[j + 3]], rows_b, sgb)
                return 0

            lax.fori_loop(0, BLKC // 2 - 1, pair, 0)
            wait_gather(rows_a, sga)
            pltpu.sync_copy(rows_a, acc.at[dst_v.at[BLKC - 2]], add=True)
            wait_gather(rows_b, sgb)
            pltpu.sync_copy(rows_b, acc.at[dst_v.at[BLKC - 1]], add=True)
            return 0

        lax.fori_loop(0, n_chunks // BLKC, stage, 0)
        plsc.subcore_barrier()
        copy_out(out_hbm)

    return pl.kernel(body, out_type=out_type, mesh=mesh, scratch_types=scratch)


_BLK = 1000


def _proj2_body(x_ref, wa_ref, wb_ref, oa_ref, ob_ref):
    x = x_ref[...]
    oa_ref[...] = jnp.dot(x, wa_ref[...], preferred_element_type=jnp.float32)
    ob_ref[...] = jnp.dot(x, wb_ref[...], preferred_element_type=jnp.float32)


def _proj2(x, wa, wb):
    n, d = x.shape
    h = wa.shape[1]
    return pl.pallas_call(
        _proj2_body,
        grid=(n // _BLK,),
        in_specs=[
            pl.BlockSpec((_BLK, d), lambda i: (i, 0)),
            pl.BlockSpec((d, h), lambda i: (0, 0)),
            pl.BlockSpec((d, h), lambda i: (0, 0)),
        ],
        out_specs=[
            pl.BlockSpec((_BLK, h), lambda i: (i, 0)),
            pl.BlockSpec((_BLK, h), lambda i: (i, 0)),
        ],
        out_shape=[jax.ShapeDtypeStruct((n, h), jnp.float32)] * 2,
    )(x, wa, wb)


def _mid_body(s1_ref, p_ref, degp_ref, b1_ref, wa_ref, wb_ref, s2_ref, y2_ref):
    deg = degp_ref[0, :, 0] + degp_ref[1, :, 0]
    inv = 1.0 / jnp.maximum(deg, 1.0)
    agg = (p_ref[0] + p_ref[1]) * inv[:, None]
    h1 = jnp.maximum(s1_ref[...] + agg + b1_ref[...], 0.0)
    s2_ref[...] = jnp.dot(h1, wa_ref[...], preferred_element_type=jnp.float32)
    y2_ref[...] = jnp.dot(h1, wb_ref[...], preferred_element_type=jnp.float32)


def _mid(s1, p, degp, b1, wa, wb):
    n, h = s1.shape
    return pl.pallas_call(
        _mid_body,
        grid=(n // _BLK,),
        in_specs=[
            pl.BlockSpec((_BLK, h), lambda i: (i, 0)),
            pl.BlockSpec((NC, _BLK, h), lambda i: (0, i, 0)),
            pl.BlockSpec((NC, _BLK, h), lambda i: (0, i, 0)),
            pl.BlockSpec((1, h), lambda i: (0, 0)),
            pl.BlockSpec((h, h), lambda i: (0, 0)),
            pl.BlockSpec((h, h), lambda i: (0, 0)),
        ],
        out_specs=[
            pl.BlockSpec((_BLK, h), lambda i: (i, 0)),
            pl.BlockSpec((_BLK, h), lambda i: (i, 0)),
        ],
        out_shape=[jax.ShapeDtypeStruct((n, h), jnp.float32)] * 2,
    )(s1, p, degp, b1[None, :], wa, wb)


def _fin_body(s2_ref, q_ref, degp_ref, b2_ref, out_ref):
    deg = degp_ref[0, :, 0] + degp_ref[1, :, 0]
    inv = 1.0 / jnp.maximum(deg, 1.0)
    out_ref[...] = s2_ref[...] + (q_ref[0] + q_ref[1]) * inv[:, None] + b2_ref[...]


def _fin(s2, q, degp, b2):
    n, h = s2.shape
    return pl.pallas_call(
        _fin_body,
        grid=(n // _BLK,),
        in_specs=[
            pl.BlockSpec((_BLK, h), lambda i: (i, 0)),
            pl.BlockSpec((NC, _BLK, h), lambda i: (0, i, 0)),
            pl.BlockSpec((NC, _BLK, h), lambda i: (0, i, 0)),
            pl.BlockSpec((1, h), lambda i: (0, 0)),
        ],
        out_specs=pl.BlockSpec((_BLK, h), lambda i: (i, 0)),
        out_shape=jax.ShapeDtypeStruct((n, h), jnp.float32),
    )(s2, q, degp, b2[None, :])


def kernel(in_feat, edge_index, W_self1, W_neigh1, b1, W_self2, W_neigh2, b2):
    n, d = in_feat.shape
    e = edge_index.shape[1]
    ept = NW * CHUNK
    n_chunks = -(-e // ept)
    n_chunks = -(-n_chunks // BLKC) * BLKC
    e_pad = n_chunks * ept

    src = jnp.pad(edge_index[0], (0, e_pad - e))
    dst = jnp.pad(edge_index[1], (0, e_pad - e), constant_values=n)
    srcr = src.reshape(NW, n_chunks, CHUNK)
    dstr = dst.reshape(NW, n_chunks, CHUNK)

    s1, y1 = _proj2(in_feat, W_self1, W_neigh1)
    p, degp = _edge_agg(n, d, n_chunks, True)(y1, srcr, dstr)
    s2, y2 = _mid(s1, p, degp, b1, W_self2, W_neigh2)
    (q,) = _edge_agg(n, d, n_chunks, False)(y2, srcr, dstr)
    return _fin(s2, q, degp, b2)


# speedup vs baseline: 3.2925x; 1.0049x over previous
import functools

import jax
import jax.numpy as jnp
from jax import lax
from jax.experimental import pallas as pl
from jax.experimental.pallas import tpu as pltpu
from jax.experimental.pallas import tpu_sc as plsc

NC = 2
NS = 16
NW = NC * NS
LANES = 16
CHUNK = 128
SPLIT = 4
SUB = CHUNK // SPLIT
BLKC = 16


@functools.lru_cache(maxsize=None)
def _edge_agg(n_nodes, d, n_chunks, want_deg):
    npad = ((n_nodes + 1 + 127) // 128) * 128
    zrows = npad // NS
    assert n_chunks % BLKC == 0
    zsegs = []
    r0 = 0
    while r0 < zrows:
        zsegs.append((r0, min(CHUNK, zrows - r0)))
        r0 += CHUNK

    out_type = [jax.ShapeDtypeStruct((NC, npad, d), jnp.float32)]
    if want_deg:
        out_type.append(jax.ShapeDtypeStruct((NC, npad, d), jnp.float32))
    scratch = [
        pltpu.VMEM((BLKC, CHUNK), jnp.int32),
        pltpu.VMEM((BLKC, CHUNK), jnp.int32),
        pltpu.VMEM((CHUNK, d), jnp.float32),
        pltpu.VMEM((CHUNK, d), jnp.float32),
        pltpu.VMEM_SHARED((npad, d), jnp.float32),
        pltpu.SemaphoreType.DMA,
        pltpu.SemaphoreType.DMA,
        pltpu.SemaphoreType.DMA,
        pltpu.SemaphoreType.DMA,
    ]

    mesh = plsc.VectorSubcoreMesh(
        core_axis_name="c", subcore_axis_name="s", num_cores=NC,
        num_subcores=NS)

    def body(y_hbm, srcr_hbm, dstr_hbm, out_hbm, *rest):
        if want_deg:
            deg_hbm, src_v, dst_v, rows_a, rows_b, acc, sga, sgb, ssa, ssb = rest
        else:
            src_v, dst_v, rows_a, rows_b, acc, sga, sgb, ssa, ssb = rest
        c = lax.axis_index("c")
        s = lax.axis_index("s")
        w = c * NS + s

        def fill_a(val):
            vv = jnp.full((LANES,), val, jnp.float32)

            def _row(i, _):
                for j in range(d // LANES):
                    rows_a[i, pl.ds(j * LANES, LANES)] = vv
                return 0

            lax.fori_loop(0, CHUNK, _row, 0)

        def zero_acc():
            for (off, nr) in zsegs:
                pltpu.sync_copy(rows_a.at[pl.ds(0, nr)],
                                acc.at[pl.ds(s * zrows + off, nr)])

        def copy_out(dst_hbm):
            pltpu.sync_copy(acc.at[pl.ds(s * zrows, zrows)],
                            dst_hbm.at[c, pl.ds(s * zrows, zrows)])

        def gather_chunk(j, buf, sem):
            for k in range(SPLIT):
                pltpu.async_copy(
                    y_hbm.at[src_v.at[j, pl.ds(k * SUB, SUB)]],
                    buf.at[pl.ds(k * SUB, SUB)], sem)

        def wait_gather(buf, sem):
            for _ in range(SPLIT):
                pltpu.make_async_copy(
                    y_hbm.at[src_v.at[0, pl.ds(0, SUB)]],
                    buf.at[pl.ds(0, SUB)], sem).wait()

        def wait_scatter(buf, sem):
            pltpu.make_async_copy(buf, acc.at[dst_v.at[0]], sem).wait()

        fill_a(0.0)
        zero_acc()

        if want_deg:
            plsc.subcore_barrier()
            fill_a(1.0)

            def dstage(b, _):
                pltpu.sync_copy(dstr_hbm.at[w, pl.ds(b * BLKC, BLKC)], dst_v)

                def dgroup(g, _):
                    for k in range(8):
                        pltpu.async_copy(
                            rows_a, acc.at[dst_v.at[g * 8 + k]], ssa, add=True)
                    for k in range(8):
                        wait_scatter(rows_a, ssa)
                    return 0

                lax.fori_loop(0, BLKC // 8, dgroup, 0)
                return 0

            lax.fori_loop(0, n_chunks // BLKC, dstage, 0)
            plsc.subcore_barrier()
            copy_out(deg_hbm)
            fill_a(0.0)
            zero_acc()

        plsc.subcore_barrier()

        def stage(b, _):
            pltpu.sync_copy(srcr_hbm.at[w, pl.ds(b * BLKC, BLKC)], src_v)
            pltpu.sync_copy(dstr_hbm.at[w, pl.ds(b * BLKC, BLKC)], dst_v)
            gather_chunk(0, rows_a, sga)
            gather_chunk(1, rows_b, sgb)

            def pair(i, _):
                j = 2 * i
                wait_gather(rows_a, sga)
                pltpu.async_copy(rows_a, acc.at[dst_v.at[j]], ssa, add=True)
                wait_gather(rows_b, sgb)
                pltpu.async_copy(rows_b, acc.at[dst_v.at[j + 1]], ssb, add=True)
                wait_scatter(rows_a, ssa)
                gather_chunk(j + 2, rows_a, sga)
                wait_scatter(rows_b, ssb)
                gather_chunk(j + 3, rows_b, sgb)
                return 0

            lax.fori_loop(0, BLKC // 2 - 1, pair, 0)
            wait_gather(rows_a, sga)
            pltpu.sync_copy(rows_a, acc.at[dst_v.at[BLKC - 2]], add=True)
            wait_gather(rows_b, sgb)
            pltpu.sync_copy(rows_b, acc.at[dst_v.at[BLKC - 1]], add=True)
            return 0

        lax.fori_loop(0, n_chunks // BLKC, stage, 0)
        plsc.subcore_barrier()
        copy_out(out_hbm)

    return pl.kernel(body, out_type=out_type, mesh=mesh, scratch_types=scratch)


_BLK = 1000


def _proj2_body(x_ref, wa_ref, wb_ref, oa_ref, ob_ref):
    x = x_ref[...]
    oa_ref[...] = jnp.dot(x, wa_ref[...], preferred_element_type=jnp.float32)
    ob_ref[...] = jnp.dot(x, wb_ref[...], preferred_element_type=jnp.float32)


def _proj2(x, wa, wb):
    n, d = x.shape
    h = wa.shape[1]
    return pl.pallas_call(
        _proj2_body,
        grid=(n // _BLK,),
        in_specs=[
            pl.BlockSpec((_BLK, d), lambda i: (i, 0)),
            pl.BlockSpec((d, h), lambda i: (0, 0)),
            pl.BlockSpec((d, h), lambda i: (0, 0)),
        ],
        out_specs=[
            pl.BlockSpec((_BLK, h), lambda i: (i, 0)),
            pl.BlockSpec((_BLK, h), lambda i: (i, 0)),
        ],
        out_shape=[jax.ShapeDtypeStruct((n, h), jnp.float32)] * 2,
    )(x, wa, wb)


def _mid_body(s1_ref, p_ref, degp_ref, b1_ref, wa_ref, wb_ref, s2_ref, y2_ref):
    deg = degp_ref[0, :, 0] + degp_ref[1, :, 0]
    inv = 1.0 / jnp.maximum(deg, 1.0)
    agg = (p_ref[0] + p_ref[1]) * inv[:, None]
    h1 = jnp.maximum(s1_ref[...] + agg + b1_ref[...], 0.0)
    s2_ref[...] = jnp.dot(h1, wa_ref[...], preferred_element_type=jnp.float32)
    y2_ref[...] = jnp.dot(h1, wb_ref[...], preferred_element_type=jnp.float32)


def _mid(s1, p, degp, b1, wa, wb):
    n, h = s1.shape
    return pl.pallas_call(
        _mid_body,
        grid=(n // _BLK,),
        in_specs=[
            pl.BlockSpec((_BLK, h), lambda i: (i, 0)),
            pl.BlockSpec((NC, _BLK, h), lambda i: (0, i, 0)),
            pl.BlockSpec((NC, _BLK, h), lambda i: (0, i, 0)),
            pl.BlockSpec((1, h), lambda i: (0, 0)),
            pl.BlockSpec((h, h), lambda i: (0, 0)),
            pl.BlockSpec((h, h), lambda i: (0, 0)),
        ],
        out_specs=[
            pl.BlockSpec((_BLK, h), lambda i: (i, 0)),
            pl.BlockSpec((_BLK, h), lambda i: (i, 0)),
        ],
        out_shape=[jax.ShapeDtypeStruct((n, h), jnp.float32)] * 2,
    )(s1, p, degp, b1[None, :], wa, wb)


def _fin_body(s2_ref, q_ref, degp_ref, b2_ref, out_ref):
    deg = degp_ref[0, :, 0] + degp_ref[1, :, 0]
    inv = 1.0 / jnp.maximum(deg, 1.0)
    out_ref[...] = s2_ref[...] + (q_ref[0] + q_ref[1]) * inv[:, None] + b2_ref[...]


def _fin(s2, q, degp, b2):
    n, h = s2.shape
    return pl.pallas_call(
        _fin_body,
        grid=(n // _BLK,),
        in_specs=[
            pl.BlockSpec((_BLK, h), lambda i: (i, 0)),
            pl.BlockSpec((NC, _BLK, h), lambda i: (0, i, 0)),
            pl.BlockSpec((NC, _BLK, h), lambda i: (0, i, 0)),
            pl.BlockSpec((1, h), lambda i: (0, 0)),
        ],
        out_specs=pl.BlockSpec((_BLK, h), lambda i: (i, 0)),
        out_shape=jax.ShapeDtypeStruct((n, h), jnp.float32),
    )(s2, q, degp, b2[None, :])


def kernel(in_feat, edge_index, W_self1, W_neigh1, b1, W_self2, W_neigh2, b2):
    n, d = in_feat.shape
    e = edge_index.shape[1]
    ept = NW * CHUNK
    n_chunks = -(-e // ept)
    n_chunks = -(-n_chunks // BLKC) * BLKC
    e_pad = n_chunks * ept

    src = jnp.pad(edge_index[0], (0, e_pad - e))
    dst = jnp.pad(edge_index[1], (0, e_pad - e), constant_values=n)
    srcr = src.reshape(NW, n_chunks, CHUNK)
    dstr = dst.reshape(NW, n_chunks, CHUNK)

    s1, y1 = _proj2(in_feat, W_self1, W_neigh1)
    p, degp = _edge_agg(n, d, n_chunks, True)(y1, srcr, dstr)
    s2, y2 = _mid(s1, p, degp, b1, W_self2, W_neigh2)
    (q,) = _edge_agg(n, d, n_chunks, False)(y2, srcr, dstr)
    return _fin(s2, q, degp, b2)


# speedup vs baseline: 3.8562x; 1.1712x over previous
import functools

import jax
import jax.numpy as jnp
from jax import lax
from jax.experimental import pallas as pl
from jax.experimental.pallas import tpu as pltpu
from jax.experimental.pallas import tpu_sc as plsc

NC = 2
NS = 16
NW = NC * NS
LANES = 16
CHUNK = 128
SPLIT = 4
SUB = CHUNK // SPLIT
BLKC = 16


@functools.lru_cache(maxsize=None)
def _edge_agg(n_nodes, d, n_chunks, want_deg):
    npad = ((n_nodes + 1 + 127) // 128) * 128
    zrows = npad // NS
    assert n_chunks % BLKC == 0
    zsegs = []
    r0 = 0
    while r0 < zrows:
        zsegs.append((r0, min(CHUNK, zrows - r0)))
        r0 += CHUNK

    out_type = [jax.ShapeDtypeStruct((NC, npad, d), jnp.float32)]
    scratch = [
        pltpu.VMEM((BLKC, CHUNK), jnp.int32),
        pltpu.VMEM((BLKC, CHUNK), jnp.int32),
        pltpu.VMEM((CHUNK, d), jnp.float32),
        pltpu.VMEM((CHUNK, d), jnp.float32),
        pltpu.VMEM_SHARED((npad, d), jnp.float32),
        pltpu.SemaphoreType.DMA,
        pltpu.SemaphoreType.DMA,
        pltpu.SemaphoreType.DMA,
        pltpu.SemaphoreType.DMA,
    ]
    if want_deg:
        out_type.append(jax.ShapeDtypeStruct((NC, npad), jnp.float32))
        scratch += [
            pltpu.VMEM((CHUNK,), jnp.float32),
            pltpu.VMEM_SHARED((npad,), jnp.float32),
            pltpu.SemaphoreType.DMA,
        ]

    mesh = plsc.VectorSubcoreMesh(
        core_axis_name="c", subcore_axis_name="s", num_cores=NC,
        num_subcores=NS)

    def body(y_hbm, srcr_hbm, dstr_hbm, out_hbm, *rest):
        if want_deg:
            (deg_hbm, src_v, dst_v, rows_a, rows_b, acc, sga, sgb, ssa, ssb,
             ones1, deg1, sdg) = rest
        else:
            src_v, dst_v, rows_a, rows_b, acc, sga, sgb, ssa, ssb = rest
        c = lax.axis_index("c")
        s = lax.axis_index("s")
        w = c * NS + s

        def fill_a(val):
            vv = jnp.full((LANES,), val, jnp.float32)

            def _row(i, _):
                for j in range(d // LANES):
                    rows_a[i, pl.ds(j * LANES, LANES)] = vv
                return 0

            lax.fori_loop(0, CHUNK, _row, 0)

        def zero_acc():
            for (off, nr) in zsegs:
                pltpu.sync_copy(rows_a.at[pl.ds(0, nr)],
                                acc.at[pl.ds(s * zrows + off, nr)])

        def copy_out(dst_hbm):
            pltpu.sync_copy(acc.at[pl.ds(s * zrows, zrows)],
                            dst_hbm.at[c, pl.ds(s * zrows, zrows)])

        def gather_chunk(j, buf, sem):
            for k in range(SPLIT):
                pltpu.async_copy(
                    y_hbm.at[src_v.at[j, pl.ds(k * SUB, SUB)]],
                    buf.at[pl.ds(k * SUB, SUB)], sem)

        def wait_gather(buf, sem):
            for _ in range(SPLIT):
                pltpu.make_async_copy(
                    y_hbm.at[src_v.at[0, pl.ds(0, SUB)]],
                    buf.at[pl.ds(0, SUB)], sem).wait()

        def wait_scatter(buf, sem):
            pltpu.make_async_copy(buf, acc.at[dst_v.at[0]], sem).wait()

        fill_a(0.0)
        zero_acc()

        if want_deg:
            zv = jnp.zeros((LANES,), jnp.float32)
            for k in range(CHUNK // LANES):
                ones1[pl.ds(k * LANES, LANES)] = zv
            r0 = 0
            while r0 < zrows:
                nr = min(CHUNK, zrows - r0)
                pltpu.sync_copy(ones1.at[pl.ds(0, nr)],
                                deg1.at[pl.ds(s * zrows + r0, nr)])
                r0 += nr
            ov = jnp.ones((LANES,), jnp.float32)
            for k in range(CHUNK // LANES):
                ones1[pl.ds(k * LANES, LANES)] = ov

        plsc.subcore_barrier()

        def stage(b, _):
            pltpu.sync_copy(srcr_hbm.at[w, pl.ds(b * BLKC, BLKC)], src_v)
            pltpu.sync_copy(dstr_hbm.at[w, pl.ds(b * BLKC, BLKC)], dst_v)
            gather_chunk(0, rows_a, sga)
            gather_chunk(1, rows_b, sgb)

            def pair(i, _):
                j = 2 * i
                wait_gather(rows_a, sga)
                pltpu.async_copy(rows_a, acc.at[dst_v.at[j]], ssa, add=True)
                if want_deg:
                    pltpu.async_copy(ones1, deg1.at[dst_v.at[j]], sdg,
                                     add=True)
                wait_gather(rows_b, sgb)
                pltpu.async_copy(rows_b, acc.at[dst_v.at[j + 1]], ssb, add=True)
                if want_deg:
                    pltpu.async_copy(ones1, deg1.at[dst_v.at[j + 1]], sdg,
                                     add=True)
                wait_scatter(rows_a, ssa)
                gather_chunk(j + 2, rows_a, sga)
                wait_scatter(rows_b, ssb)
                gather_chunk(j + 3, rows_b, sgb)
                if want_deg:
                    for _k in range(2):
                        pltpu.make_async_copy(
                            ones1, deg1.at[dst_v.at[0]], sdg).wait()
                return 0

            lax.fori_loop(0, BLKC // 2 - 1, pair, 0)
            wait_gather(rows_a, sga)
            pltpu.sync_copy(rows_a, acc.at[dst_v.at[BLKC - 2]], add=True)
            wait_gather(rows_b, sgb)
            pltpu.sync_copy(rows_b, acc.at[dst_v.at[BLKC - 1]], add=True)
            if want_deg:
                pltpu.sync_copy(ones1, deg1.at[dst_v.at[BLKC - 2]], add=True)
                pltpu.sync_copy(ones1, deg1.at[dst_v.at[BLKC - 1]], add=True)
            return 0

        lax.fori_loop(0, n_chunks // BLKC, stage, 0)
        plsc.subcore_barrier()
        copy_out(out_hbm)
        if want_deg:
            @pl.when(s == 0)
            def _():
                pltpu.sync_copy(deg1, deg_hbm.at[c])

    return pl.kernel(body, out_type=out_type, mesh=mesh, scratch_types=scratch)


def _proj2_body(x_ref, wa_ref, wb_ref, oa_ref, ob_ref):
    x = x_ref[...]
    oa_ref[...] = jnp.dot(x, wa_ref[...], preferred_element_type=jnp.float32)
    ob_ref[...] = jnp.dot(x, wb_ref[...], preferred_element_type=jnp.float32)


def _proj2(x, wa, wb, blk):
    n, d = x.shape
    h = wa.shape[1]
    grid = -(-n // blk)
    return pl.pallas_call(
        _proj2_body,
        grid=(grid,),
        in_specs=[
            pl.BlockSpec((blk, d), lambda i: (i, 0)),
            pl.BlockSpec((d, h), lambda i: (0, 0)),
            pl.BlockSpec((d, h), lambda i: (0, 0)),
        ],
        out_specs=[
            pl.BlockSpec((blk, h), lambda i: (i, 0)),
            pl.BlockSpec((blk, h), lambda i: (i, 0)),
        ],
        out_shape=[jax.ShapeDtypeStruct((n, h), jnp.float32)] * 2,
    )(x, wa, wb)


def _mid_body(s1_ref, p_ref, degp_ref, b1_ref, wa_ref, wb_ref, s2_ref, y2_ref):
    deg = degp_ref[0, 0, 0, :] + degp_ref[1, 0, 0, :]
    inv = 1.0 / jnp.maximum(deg, 1.0)
    agg = (p_ref[0] + p_ref[1]) * inv[:, None]
    h1 = jnp.maximum(s1_ref[...] + agg + b1_ref[...], 0.0)
    s2_ref[...] = jnp.dot(h1, wa_ref[...], preferred_element_type=jnp.float32)
    y2_ref[...] = jnp.dot(h1, wb_ref[...], preferred_element_type=jnp.float32)


def _mid(s1, p, degp, b1, wa, wb):
    n, h = s1.shape
    blk = degp.shape[3]
    grid = degp.shape[1]
    return pl.pallas_call(
        _mid_body,
        grid=(grid,),
        in_specs=[
            pl.BlockSpec((blk, h), lambda i: (i, 0)),
            pl.BlockSpec((NC, blk, h), lambda i: (0, i, 0)),
            pl.BlockSpec((NC, 1, 1, blk), lambda i: (0, i, 0, 0)),
            pl.BlockSpec((1, h), lambda i: (0, 0)),
            pl.BlockSpec((h, h), lambda i: (0, 0)),
            pl.BlockSpec((h, h), lambda i: (0, 0)),
        ],
        out_specs=[
            pl.BlockSpec((blk, h), lambda i: (i, 0)),
            pl.BlockSpec((blk, h), lambda i: (i, 0)),
        ],
        out_shape=[jax.ShapeDtypeStruct((n, h), jnp.float32)] * 2,
    )(s1, p, degp, b1[None, :], wa, wb)


def _fin_body(s2_ref, q_ref, degp_ref, b2_ref, out_ref):
    deg = degp_ref[0, 0, 0, :] + degp_ref[1, 0, 0, :]
    inv = 1.0 / jnp.maximum(deg, 1.0)
    out_ref[...] = s2_ref[...] + (q_ref[0] + q_ref[1]) * inv[:, None] + b2_ref[...]


def _fin(s2, q, degp, b2):
    n, h = s2.shape
    blk = degp.shape[3]
    grid = degp.shape[1]
    return pl.pallas_call(
        _fin_body,
        grid=(grid,),
        in_specs=[
            pl.BlockSpec((blk, h), lambda i: (i, 0)),
            pl.BlockSpec((NC, blk, h), lambda i: (0, i, 0)),
            pl.BlockSpec((NC, 1, 1, blk), lambda i: (0, i, 0, 0)),
            pl.BlockSpec((1, h), lambda i: (0, 0)),
        ],
        out_specs=pl.BlockSpec((blk, h), lambda i: (i, 0)),
        out_shape=jax.ShapeDtypeStruct((n, h), jnp.float32),
    )(s2, q, degp, b2[None, :])


def kernel(in_feat, edge_index, W_self1, W_neigh1, b1, W_self2, W_neigh2, b2):
    n, d = in_feat.shape
    e = edge_index.shape[1]
    ept = NW * CHUNK
    n_chunks = -(-e // ept)
    n_chunks = -(-n_chunks // BLKC) * BLKC
    e_pad = n_chunks * ept
    blk = (((n + 1 + 127) // 128) * 128) // NS

    src = jnp.pad(edge_index[0], (0, e_pad - e))
    dst = jnp.pad(edge_index[1], (0, e_pad - e), constant_values=n)
    srcr = src.reshape(NW, n_chunks, CHUNK)
    dstr = dst.reshape(NW, n_chunks, CHUNK)

    s1, y1 = _proj2(in_feat, W_self1, W_neigh1, blk)
    p, degp = _edge_agg(n, d, n_chunks, True)(y1, srcr, dstr)
    degp = degp.reshape(NC, NS, 1, blk)
    s2, y2 = _mid(s1, p, degp, b1, W_self2, W_neigh2)
    (q,) = _edge_agg(n, d, n_chunks, False)(y2, srcr, dstr)
    return _fin(s2, q, degp, b2)


# speedup vs baseline: 3.9120x; 1.0145x over previous
import functools

import jax
import jax.numpy as jnp
from jax import lax
from jax.experimental import pallas as pl
from jax.experimental.pallas import tpu as pltpu
from jax.experimental.pallas import tpu_sc as plsc

NC = 2
NS = 16
NW = NC * NS
LANES = 16
CHUNK = 128
SPLIT = 4
SUB = CHUNK // SPLIT
BLKC = 40


@functools.lru_cache(maxsize=None)
def _edge_agg(n_nodes, d, n_chunks, want_deg):
    npad = ((n_nodes + 1 + 127) // 128) * 128
    zrows = npad // NS
    assert n_chunks % BLKC == 0
    zsegs = []
    r0 = 0
    while r0 < zrows:
        zsegs.append((r0, min(CHUNK, zrows - r0)))
        r0 += CHUNK

    out_type = [jax.ShapeDtypeStruct((NC, npad, d), jnp.float32)]
    scratch = [
        pltpu.VMEM((BLKC, CHUNK), jnp.int32),
        pltpu.VMEM((BLKC, CHUNK), jnp.int32),
        pltpu.VMEM((CHUNK, d), jnp.float32),
        pltpu.VMEM((CHUNK, d), jnp.float32),
        pltpu.VMEM_SHARED((npad, d), jnp.float32),
        pltpu.SemaphoreType.DMA,
        pltpu.SemaphoreType.DMA,
        pltpu.SemaphoreType.DMA,
        pltpu.SemaphoreType.DMA,
    ]
    if want_deg:
        out_type.append(jax.ShapeDtypeStruct((NC, npad), jnp.float32))
        scratch += [
            pltpu.VMEM((CHUNK,), jnp.float32),
            pltpu.VMEM_SHARED((npad,), jnp.float32),
            pltpu.SemaphoreType.DMA,
        ]

    mesh = plsc.VectorSubcoreMesh(
        core_axis_name="c", subcore_axis_name="s", num_cores=NC,
        num_subcores=NS)

    def body(y_hbm, srcr_hbm, dstr_hbm, out_hbm, *rest):
        if want_deg:
            (deg_hbm, src_v, dst_v, rows_a, rows_b, acc, sga, sgb, ssa, ssb,
             ones1, deg1, sdg) = rest
        else:
            src_v, dst_v, rows_a, rows_b, acc, sga, sgb, ssa, ssb = rest
        c = lax.axis_index("c")
        s = lax.axis_index("s")
        w = c * NS + s

        def fill_a(val):
            vv = jnp.full((LANES,), val, jnp.float32)

            def _row(i, _):
                for j in range(d // LANES):
                    rows_a[i, pl.ds(j * LANES, LANES)] = vv
                return 0

            lax.fori_loop(0, CHUNK, _row, 0)

        def zero_acc():
            for (off, nr) in zsegs:
                pltpu.sync_copy(rows_a.at[pl.ds(0, nr)],
                                acc.at[pl.ds(s * zrows + off, nr)])

        def copy_out(dst_hbm):
            pltpu.sync_copy(acc.at[pl.ds(s * zrows, zrows)],
                            dst_hbm.at[c, pl.ds(s * zrows, zrows)])

        def gather_chunk(j, buf, sem):
            for k in range(SPLIT):
                pltpu.async_copy(
                    y_hbm.at[src_v.at[j, pl.ds(k * SUB, SUB)]],
                    buf.at[pl.ds(k * SUB, SUB)], sem)

        def wait_gather(buf, sem):
            for _ in range(SPLIT):
                pltpu.make_async_copy(
                    y_hbm.at[src_v.at[0, pl.ds(0, SUB)]],
                    buf.at[pl.ds(0, SUB)], sem).wait()

        def wait_scatter(buf, sem):
            pltpu.make_async_copy(buf, acc.at[dst_v.at[0]], sem).wait()

        fill_a(0.0)
        zero_acc()

        if want_deg:
            zv = jnp.zeros((LANES,), jnp.float32)
            for k in range(CHUNK // LANES):
                ones1[pl.ds(k * LANES, LANES)] = zv
            r0 = 0
            while r0 < zrows:
                nr = min(CHUNK, zrows - r0)
                pltpu.sync_copy(ones1.at[pl.ds(0, nr)],
                                deg1.at[pl.ds(s * zrows + r0, nr)])
                r0 += nr
            ov = jnp.ones((LANES,), jnp.float32)
            for k in range(CHUNK // LANES):
                ones1[pl.ds(k * LANES, LANES)] = ov

        plsc.subcore_barrier()

        def stage(b, _):
            pltpu.sync_copy(srcr_hbm.at[w, pl.ds(b * BLKC, BLKC)], src_v)
            pltpu.sync_copy(dstr_hbm.at[w, pl.ds(b * BLKC, BLKC)], dst_v)
            gather_chunk(0, rows_a, sga)
            gather_chunk(1, rows_b, sgb)

            def pair(i, _):
                j = 2 * i
                wait_gather(rows_a, sga)
                pltpu.async_copy(rows_a, acc.at[dst_v.at[j]], ssa, add=True)
                if want_deg:
                    pltpu.async_copy(ones1, deg1.at[dst_v.at[j]], sdg,
                                     add=True)
                wait_gather(rows_b, sgb)
                pltpu.async_copy(rows_b, acc.at[dst_v.at[j + 1]], ssb, add=True)
                if want_deg:
                    pltpu.async_copy(ones1, deg1.at[dst_v.at[j + 1]], sdg,
                                     add=True)
                wait_scatter(rows_a, ssa)
                gather_chunk(j + 2, rows_a, sga)
                wait_scatter(rows_b, ssb)
                gather_chunk(j + 3, rows_b, sgb)
                if want_deg:
                    for _k in range(2):
                        pltpu.make_async_copy(
                            ones1, deg1.at[dst_v.at[0]], sdg).wait()
                return 0

            lax.fori_loop(0, BLKC // 2 - 1, pair, 0)
            wait_gather(rows_a, sga)
            pltpu.sync_copy(rows_a, acc.at[dst_v.at[BLKC - 2]], add=True)
            wait_gather(rows_b, sgb)
            pltpu.sync_copy(rows_b, acc.at[dst_v.at[BLKC - 1]], add=True)
            if want_deg:
                pltpu.sync_copy(ones1, deg1.at[dst_v.at[BLKC - 2]], add=True)
                pltpu.sync_copy(ones1, deg1.at[dst_v.at[BLKC - 1]], add=True)
            return 0

        lax.fori_loop(0, n_chunks // BLKC, stage, 0)
        plsc.subcore_barrier()
        copy_out(out_hbm)
        if want_deg:
            @pl.when(s == 0)
            def _():
                pltpu.sync_copy(deg1, deg_hbm.at[c])

    return pl.kernel(body, out_type=out_type, mesh=mesh, scratch_types=scratch)


def _proj2_body(x_ref, wa_ref, wb_ref, oa_ref, ob_ref):
    x = x_ref[...]
    oa_ref[...] = jnp.dot(x, wa_ref[...], preferred_element_type=jnp.float32)
    ob_ref[...] = jnp.dot(x, wb_ref[...], preferred_element_type=jnp.float32)


def _proj2(x, wa, wb, blk):
    n, d = x.shape
    h = wa.shape[1]
    grid = -(-n // blk)
    return pl.pallas_call(
        _proj2_body,
        grid=(grid,),
        in_specs=[
            pl.BlockSpec((blk, d), lambda i: (i, 0)),
            pl.BlockSpec((d, h), lambda i: (0, 0)),
            pl.BlockSpec((d, h), lambda i: (0, 0)),
        ],
        out_specs=[
            pl.BlockSpec((blk, h), lambda i: (i, 0)),
            pl.BlockSpec((blk, h), lambda i: (i, 0)),
        ],
        out_shape=[jax.ShapeDtypeStruct((n, h), jnp.float32)] * 2,
    )(x, wa, wb)


def _mid_body(s1_ref, p_ref, degp_ref, b1_ref, wa_ref, wb_ref, s2_ref, y2_ref):
    deg = degp_ref[0, 0, 0, :] + degp_ref[1, 0, 0, :]
    inv = 1.0 / jnp.maximum(deg, 1.0)
    agg = (p_ref[0] + p_ref[1]) * inv[:, None]
    h1 = jnp.maximum(s1_ref[...] + agg + b1_ref[...], 0.0)
    s2_ref[...] = jnp.dot(h1, wa_ref[...], preferred_element_type=jnp.float32)
    y2_ref[...] = jnp.dot(h1, wb_ref[...], preferred_element_type=jnp.float32)


def _mid(s1, p, degp, b1, wa, wb):
    n, h = s1.shape
    blk = degp.shape[3]
    grid = degp.shape[1]
    return pl.pallas_call(
        _mid_body,
        grid=(grid,),
        in_specs=[
            pl.BlockSpec((blk, h), lambda i: (i, 0)),
            pl.BlockSpec((NC, blk, h), lambda i: (0, i, 0)),
            pl.BlockSpec((NC, 1, 1, blk), lambda i: (0, i, 0, 0)),
            pl.BlockSpec((1, h), lambda i: (0, 0)),
            pl.BlockSpec((h, h), lambda i: (0, 0)),
            pl.BlockSpec((h, h), lambda i: (0, 0)),
        ],
        out_specs=[
            pl.BlockSpec((blk, h), lambda i: (i, 0)),
            pl.BlockSpec((blk, h), lambda i: (i, 0)),
        ],
        out_shape=[jax.ShapeDtypeStruct((n, h), jnp.float32)] * 2,
    )(s1, p, degp, b1[None, :], wa, wb)


def _fin_body(s2_ref, q_ref, degp_ref, b2_ref, out_ref):
    deg = degp_ref[0, 0, 0, :] + degp_ref[1, 0, 0, :]
    inv = 1.0 / jnp.maximum(deg, 1.0)
    out_ref[...] = s2_ref[...] + (q_ref[0] + q_ref[1]) * inv[:, None] + b2_ref[...]


def _fin(s2, q, degp, b2):
    n, h = s2.shape
    blk = degp.shape[3]
    grid = degp.shape[1]
    return pl.pallas_call(
        _fin_body,
        grid=(grid,),
        in_specs=[
            pl.BlockSpec((blk, h), lambda i: (i, 0)),
            pl.BlockSpec((NC, blk, h), lambda i: (0, i, 0)),
            pl.BlockSpec((NC, 1, 1, blk), lambda i: (0, i, 0, 0)),
            pl.BlockSpec((1, h), lambda i: (0, 0)),
        ],
        out_specs=pl.BlockSpec((blk, h), lambda i: (i, 0)),
        out_shape=jax.ShapeDtypeStruct((n, h), jnp.float32),
    )(s2, q, degp, b2[None, :])


def kernel(in_feat, edge_index, W_self1, W_neigh1, b1, W_self2, W_neigh2, b2):
    n, d = in_feat.shape
    e = edge_index.shape[1]
    ept = NW * CHUNK
    n_chunks = -(-e // ept)
    n_chunks = -(-n_chunks // BLKC) * BLKC
    e_pad = n_chunks * ept
    blk = (((n + 1 + 127) // 128) * 128) // NS

    src = jnp.pad(edge_index[0], (0, e_pad - e))
    dst = jnp.pad(edge_index[1], (0, e_pad - e), constant_values=n)
    srcr = src.reshape(NW, n_chunks, CHUNK)
    dstr = dst.reshape(NW, n_chunks, CHUNK)

    s1, y1 = _proj2(in_feat, W_self1, W_neigh1, blk)
    p, degp = _edge_agg(n, d, n_chunks, True)(y1, srcr, dstr)
    degp = degp.reshape(NC, NS, 1, blk)
    s2, y2 = _mid(s1, p, degp, b1, W_self2, W_neigh2)
    (q,) = _edge_agg(n, d, n_chunks, False)(y2, srcr, dstr)
    return _fin(s2, q, degp, b2)
